# Initial kernel scaffold; baseline (speedup 1.0000x reference)
import jax
import jax.numpy as jnp
from jax.experimental import pallas as pl


def kernel(image):
    raise NotImplementedError("write your pallas kernel here")


# speedup vs baseline: 1.0376x; 1.0376x over previous
import numpy as np
import jax
import jax.numpy as jnp
from jax import lax
from jax.experimental import pallas as pl
from jax.experimental.pallas import tpu as pltpu

_PATCH = 32
_NPH = 16
_NUM_PATCHES = _NPH * _NPH
_NUM_MASKED = 154
_C_BLK = 8

_perm = np.asarray(jax.random.permutation(jax.random.key(42), _NUM_PATCHES))
_idx_pad = np.full((_NUM_PATCHES,), -1, dtype=np.int32)
_idx_pad[:_NUM_MASKED] = _perm[:_NUM_MASKED].astype(np.int32)
_IDX_ROW = _idx_pad.reshape(1, _NUM_PATCHES)
_IDX_COL = _idx_pad.reshape(_NUM_PATCHES, 1)


def _mask_kernel(idx_row_ref, idx_col_ref, keep_ref, maskb_ref):
    n = _NUM_PATCHES
    hh = lax.broadcasted_iota(jnp.int32, (512, n), 0) // _PATCH
    ph = idx_row_ref[...] // _NPH
    r = (hh == ph).astype(jnp.float32)
    ww = lax.broadcasted_iota(jnp.int32, (n, 512), 1) // _PATCH
    pw = idx_col_ref[...] % _NPH
    c = (ww == pw).astype(jnp.float32)
    mask = jnp.dot(r, c, preferred_element_type=jnp.float32)
    keep_ref[...] = 1.0 - mask
    maskb_ref[...] = (mask > 0.5)[None, :, :]


def _mul_kernel(img_ref, keep_ref, out_ref):
    out_ref[...] = img_ref[...] * keep_ref[...][None, :, :]


def kernel(image):
    C, H, W = image.shape
    keep, mask_full = pl.pallas_call(
        _mask_kernel,
        out_shape=(
            jax.ShapeDtypeStruct((H, W), jnp.float32),
            jax.ShapeDtypeStruct((1, H, W), jnp.bool_),
        ),
    )(jnp.asarray(_IDX_ROW), jnp.asarray(_IDX_COL))

    masked = pl.pallas_call(
        _mul_kernel,
        grid=(C // _C_BLK,),
        in_specs=[
            pl.BlockSpec((_C_BLK, H, W), lambda i: (i, 0, 0)),
            pl.BlockSpec((H, W), lambda i: (0, 0)),
        ],
        out_specs=pl.BlockSpec((_C_BLK, H, W), lambda i: (i, 0, 0)),
        out_shape=jax.ShapeDtypeStruct((C, H, W), jnp.float32),
        compiler_params=pltpu.CompilerParams(
            dimension_semantics=("parallel",),
        ),
    )(image, keep)
    return masked, mask_full
